# Initial kernel scaffold; baseline (speedup 1.0000x reference)
import functools

import jax
import jax.numpy as jnp
from jax import lax
from jax.experimental import pallas as pl
from jax.experimental.pallas import tpu as pltpu
from jax.experimental.pallas import tpu_sc as plsc

NC = 2
NS = 16
L = 16

CH = 80

_HIGH = jax.lax.Precision.HIGHEST


def _sc_mesh():
    return plsc.VectorSubcoreMesh(core_axis_name="c", subcore_axis_name="s")


def _sc_degree_hist(dst, n):
    (e,) = dst.shape
    epc = e // NC
    ept = epc // NS
    nch = ept // CH
    rpt = n // NS
    zr = 125
    assert epc * NC == e and ept * NS == epc and nch * CH == ept
    assert rpt * NS == n and rpt % zr == 0

    @functools.partial(
        pl.kernel,
        mesh=_sc_mesh(),
        out_type=jax.ShapeDtypeStruct((NC, n, L), jnp.float32),
        scratch_types=[
            pltpu.VMEM((CH,), jnp.int32),
            pltpu.VMEM((CH, L), jnp.float32),
            pltpu.VMEM((zr, L), jnp.float32),
            pltpu.VMEM_SHARED((n, L), jnp.float32),
        ],
    )
    def k(dst_hbm, out_hbm, idx_v, ones_v, z_v, acc_sh):
        cid = lax.axis_index("c")
        sid = lax.axis_index("s")

        @pl.loop(0, CH)
        def _(r):
            ones_v[r, :] = jnp.full((L,), 1.0, jnp.float32)

        @pl.loop(0, zr)
        def _(r):
            z_v[r, :] = jnp.zeros((L,), jnp.float32)

        base = sid * rpt

        @pl.loop(0, rpt // zr)
        def _(j):
            pltpu.sync_copy(z_v, acc_sh.at[pl.ds(base + j * zr, zr)])

        plsc.subcore_barrier()

        ebase = cid * epc + sid * ept

        @pl.loop(0, nch)
        def _(j):
            pltpu.sync_copy(dst_hbm.at[pl.ds(ebase + j * CH, CH)], idx_v)
            pltpu.sync_copy(ones_v, acc_sh.at[idx_v], add=True)

        plsc.subcore_barrier()
        pltpu.sync_copy(acc_sh.at[pl.ds(base, rpt)],
                        out_hbm.at[cid, pl.ds(base, rpt)])

    return k(dst)


def _sc_aggregate(u, src, dst):
    n, d = u.shape
    (e,) = src.shape
    epc = e // NC
    ept = epc // NS
    nch = ept // CH
    rpt = n // NS
    zr = 125
    assert epc * NC == e and ept * NS == epc and nch * CH == ept
    assert rpt * NS == n and rpt % zr == 0 and d % L == 0

    @functools.partial(
        pl.kernel,
        mesh=_sc_mesh(),
        out_type=jax.ShapeDtypeStruct((NC, n, d), jnp.float32),
        scratch_types=[
            pltpu.VMEM((CH,), jnp.int32),
            pltpu.VMEM((CH,), jnp.int32),
            pltpu.VMEM((CH, d), jnp.float32),
            pltpu.VMEM((zr, d), jnp.float32),
            pltpu.VMEM_SHARED((n, d), jnp.float32),
            pltpu.SemaphoreType.DMA,
        ],
    )
    def k(u_hbm, src_hbm, dst_hbm, out_hbm, si_v, di_v, rows_v, z_v, acc_sh,
          sem):
        cid = lax.axis_index("c")
        sid = lax.axis_index("s")

        @pl.loop(0, zr)
        def _(r):
            @pl.loop(0, d, step=L)
            def _(c):
                z_v[r, pl.ds(c, L)] = jnp.zeros((L,), jnp.float32)

        base = sid * rpt

        @pl.loop(0, rpt // zr)
        def _(j):
            pltpu.sync_copy(z_v, acc_sh.at[pl.ds(base + j * zr, zr)])

        plsc.subcore_barrier()

        ebase = cid * epc + sid * ept

        @pl.loop(0, nch)
        def _(j):
            pltpu.sync_copy(src_hbm.at[pl.ds(ebase + j * CH, CH)], si_v)
            pltpu.sync_copy(dst_hbm.at[pl.ds(ebase + j * CH, CH)], di_v)
            pltpu.async_copy(u_hbm.at[si_v], rows_v, sem).wait()
            pltpu.sync_copy(rows_v, acc_sh.at[di_v], add=True)

        plsc.subcore_barrier()
        pltpu.sync_copy(acc_sh.at[pl.ds(base, rpt)],
                        out_hbm.at[cid, pl.ds(base, rpt)])

    return k(u, src, dst)


def _tc_prep(deg_p, x, bn=1000):
    n, d = x.shape

    def body(dp_ref, x_ref, dinv_ref, u_ref):
        deg = dp_ref[0, :, 0:1] + dp_ref[1, :, 0:1] + 1.0
        dinv = lax.rsqrt(deg)
        dinv_ref[...] = dinv
        u_ref[...] = x_ref[...] * dinv

    return pl.pallas_call(
        body,
        grid=(n // bn,),
        in_specs=[
            pl.BlockSpec((NC, bn, L), lambda i: (0, i, 0)),
            pl.BlockSpec((bn, d), lambda i: (i, 0)),
        ],
        out_specs=[
            pl.BlockSpec((bn, 1), lambda i: (i, 0)),
            pl.BlockSpec((bn, d), lambda i: (i, 0)),
        ],
        out_shape=[
            jax.ShapeDtypeStruct((n, 1), jnp.float32),
            jax.ShapeDtypeStruct((n, d), jnp.float32),
        ],
    )(deg_p, x)


def _tc_layer1(s_p, u, dinv, w_local, b_local, w_g1, b_g1, w_f1, b_f,
               bn=1000):
    n, d = u.shape

    def body(s_ref, u_ref, dv_ref, wl, bl, wg, bg, wf, bf, pre_ref, u2_ref):
        y = dv_ref[...] * (s_ref[0] + s_ref[1] + u_ref[...])
        loc = jnp.maximum(
            jnp.dot(y, wl[...], precision=_HIGH) + bl[...], 0.0)
        pre_ref[...] = jnp.dot(loc, wf[...], precision=_HIGH) + bf[...]
        g1 = jnp.maximum(
            jnp.dot(y, wg[...], precision=_HIGH) + bg[...], 0.0)
        u2_ref[...] = dv_ref[...] * g1

    full = lambda i: (0, 0)
    return pl.pallas_call(
        body,
        grid=(n // bn,),
        in_specs=[
            pl.BlockSpec((NC, bn, d), lambda i: (0, i, 0)),
            pl.BlockSpec((bn, d), lambda i: (i, 0)),
            pl.BlockSpec((bn, 1), lambda i: (i, 0)),
            pl.BlockSpec((d, d), full),
            pl.BlockSpec((1, d), full),
            pl.BlockSpec((d, d), full),
            pl.BlockSpec((1, d), full),
            pl.BlockSpec((d, d), full),
            pl.BlockSpec((1, d), full),
        ],
        out_specs=[
            pl.BlockSpec((bn, d), lambda i: (i, 0)),
            pl.BlockSpec((bn, d), lambda i: (i, 0)),
        ],
        out_shape=[
            jax.ShapeDtypeStruct((n, d), jnp.float32),
            jax.ShapeDtypeStruct((n, d), jnp.float32),
        ],
    )(s_p, u, dinv, w_local, b_local, w_g1, b_g1, w_f1, b_f)


def _tc_layer2(s_p, u2, dinv, w_g2, b_g2, w_f2, pre, bn=1000):
    n, d = u2.shape

    def body(s_ref, u2_ref, dv_ref, wg, bg, wf, pre_ref, out_ref):
        z = dv_ref[...] * (s_ref[0] + s_ref[1] + u2_ref[...])
        g2 = jnp.maximum(
            jnp.dot(z, wg[...], precision=_HIGH) + bg[...], 0.0)
        out_ref[...] = pre_ref[...] + jnp.dot(g2, wf[...], precision=_HIGH)

    full = lambda i: (0, 0)
    return pl.pallas_call(
        body,
        grid=(n // bn,),
        in_specs=[
            pl.BlockSpec((NC, bn, d), lambda i: (0, i, 0)),
            pl.BlockSpec((bn, d), lambda i: (i, 0)),
            pl.BlockSpec((bn, 1), lambda i: (i, 0)),
            pl.BlockSpec((d, d), full),
            pl.BlockSpec((1, d), full),
            pl.BlockSpec((d, d), full),
            pl.BlockSpec((bn, d), lambda i: (i, 0)),
        ],
        out_specs=pl.BlockSpec((bn, d), lambda i: (i, 0)),
        out_shape=jax.ShapeDtypeStruct((n, d), jnp.float32),
    )(s_p, u2, dinv, w_g2, b_g2, w_f2, pre)


def kernel(x, edge_index, W_local, b_local, W_g1, b_g1, W_g2, b_g2, W_fuse,
           b_fuse):
    n, d = x.shape
    h = W_local.shape[1]
    src = edge_index[0]
    dst = edge_index[1]
    w_f1 = W_fuse[:h]
    w_f2 = W_fuse[h:]
    b_l = b_local.reshape(1, -1)
    b_g1r = b_g1.reshape(1, -1)
    b_g2r = b_g2.reshape(1, -1)
    b_f = b_fuse.reshape(1, -1)

    deg_p = _sc_degree_hist(dst, n)
    dinv, u = _tc_prep(deg_p, x)
    s1 = _sc_aggregate(u, src, dst)
    pre, u2 = _tc_layer1(s1, u, dinv, W_local, b_l, W_g1, b_g1r, w_f1, b_f)
    s2 = _sc_aggregate(u2, src, dst)
    return _tc_layer2(s2, u2, dinv, W_g2, b_g2r, w_f2, pre)


# speedup vs baseline: 13.4992x; 13.4992x over previous
import functools

import jax
import jax.numpy as jnp
from jax import lax
from jax.experimental import pallas as pl
from jax.experimental.pallas import tpu as pltpu
from jax.experimental.pallas import tpu_sc as plsc

NC = 2
NS = 16
L = 16

CH = 80

_HIGH = jax.lax.Precision.HIGHEST


def _sc_mesh():
    return plsc.VectorSubcoreMesh(core_axis_name="c", subcore_axis_name="s")


def _sc_degree_hist(dst, n, d=128):
    (e,) = dst.shape
    epc = e // NC
    ept = epc // NS
    nch = ept // CH
    rpt = -(-n // NS)
    rpt = -(-rpt // 128) * 128
    npad = rpt * NS
    zr = 128
    assert epc * NC == e and ept * NS == epc and nch * CH == ept

    @functools.partial(
        pl.kernel,
        mesh=_sc_mesh(),
        out_type=jax.ShapeDtypeStruct((NC, npad, d), jnp.float32),
        scratch_types=[
            pltpu.VMEM((CH,), jnp.int32),
            pltpu.VMEM((CH, d), jnp.float32),
            pltpu.VMEM((zr, d), jnp.float32),
            pltpu.VMEM_SHARED((npad, d), jnp.float32),
        ],
    )
    def k(dst_hbm, out_hbm, idx_v, ones_v, z_v, acc_sh):
        cid = lax.axis_index("c")
        sid = lax.axis_index("s")

        @pl.loop(0, CH)
        def _(r):
            @pl.loop(0, d, step=L)
            def _(c):
                ones_v[r, pl.ds(c, L)] = jnp.full((L,), 1.0, jnp.float32)

        @pl.loop(0, zr)
        def _(r):
            @pl.loop(0, d, step=L)
            def _(c):
                z_v[r, pl.ds(c, L)] = jnp.zeros((L,), jnp.float32)

        base = sid * rpt

        @pl.loop(0, rpt // zr)
        def _(j):
            pltpu.sync_copy(z_v, acc_sh.at[pl.ds(base + j * zr, zr)])

        plsc.subcore_barrier()

        ebase = cid * epc + sid * ept

        @pl.loop(0, nch)
        def _(j):
            pltpu.sync_copy(dst_hbm.at[pl.ds(ebase + j * CH, CH)], idx_v)
            pltpu.sync_copy(ones_v, acc_sh.at[idx_v], add=True)

        plsc.subcore_barrier()
        pltpu.sync_copy(acc_sh.at[pl.ds(base, rpt)],
                        out_hbm.at[cid, pl.ds(base, rpt)])

    return k(dst)


def _sc_aggregate(u, src, dst):
    n, d = u.shape
    (e,) = src.shape
    epc = e // NC
    ept = epc // NS
    nch = ept // CH
    rpt = -(-n // NS)
    rpt = -(-rpt // 128) * 128
    npad = rpt * NS
    zr = 128
    assert epc * NC == e and ept * NS == epc and nch * CH == ept
    assert d % L == 0

    @functools.partial(
        pl.kernel,
        mesh=_sc_mesh(),
        out_type=jax.ShapeDtypeStruct((NC, npad, d), jnp.float32),
        scratch_types=[
            pltpu.VMEM((CH,), jnp.int32),
            pltpu.VMEM((CH,), jnp.int32),
            pltpu.VMEM((CH, d), jnp.float32),
            pltpu.VMEM((zr, d), jnp.float32),
            pltpu.VMEM_SHARED((npad, d), jnp.float32),
            pltpu.SemaphoreType.DMA,
        ],
    )
    def k(u_hbm, src_hbm, dst_hbm, out_hbm, si_v, di_v, rows_v, z_v, acc_sh,
          sem):
        cid = lax.axis_index("c")
        sid = lax.axis_index("s")

        @pl.loop(0, zr)
        def _(r):
            @pl.loop(0, d, step=L)
            def _(c):
                z_v[r, pl.ds(c, L)] = jnp.zeros((L,), jnp.float32)

        base = sid * rpt

        @pl.loop(0, rpt // zr)
        def _(j):
            pltpu.sync_copy(z_v, acc_sh.at[pl.ds(base + j * zr, zr)])

        plsc.subcore_barrier()

        ebase = cid * epc + sid * ept

        @pl.loop(0, nch)
        def _(j):
            pltpu.sync_copy(src_hbm.at[pl.ds(ebase + j * CH, CH)], si_v)
            pltpu.sync_copy(dst_hbm.at[pl.ds(ebase + j * CH, CH)], di_v)
            pltpu.async_copy(u_hbm.at[si_v], rows_v, sem).wait()
            pltpu.sync_copy(rows_v, acc_sh.at[di_v], add=True)

        plsc.subcore_barrier()
        pltpu.sync_copy(acc_sh.at[pl.ds(base, rpt)],
                        out_hbm.at[cid, pl.ds(base, rpt)])

    return k(u, src, dst)


def _tc_prep(deg_p, x, bn=1000):
    n, d = x.shape

    def body(dp_ref, x_ref, dinv_ref, u_ref):
        deg = dp_ref[0, :, 0:1] + dp_ref[1, :, 0:1] + 1.0
        dinv = lax.rsqrt(deg)
        dinv_ref[...] = dinv
        u_ref[...] = x_ref[...] * dinv

    return pl.pallas_call(
        body,
        grid=(n // bn,),
        in_specs=[
            pl.BlockSpec((NC, bn, deg_p.shape[2]), lambda i: (0, i, 0)),
            pl.BlockSpec((bn, d), lambda i: (i, 0)),
        ],
        out_specs=[
            pl.BlockSpec((bn, 1), lambda i: (i, 0)),
            pl.BlockSpec((bn, d), lambda i: (i, 0)),
        ],
        out_shape=[
            jax.ShapeDtypeStruct((n, 1), jnp.float32),
            jax.ShapeDtypeStruct((n, d), jnp.float32),
        ],
    )(deg_p, x)


def _tc_layer1(s_p, u, dinv, w_local, b_local, w_g1, b_g1, w_f1, b_f,
               bn=1000):
    n, d = u.shape

    def body(s_ref, u_ref, dv_ref, wl, bl, wg, bg, wf, bf, pre_ref, u2_ref):
        y = dv_ref[...] * (s_ref[0] + s_ref[1] + u_ref[...])
        loc = jnp.maximum(
            jnp.dot(y, wl[...], precision=_HIGH) + bl[...], 0.0)
        pre_ref[...] = jnp.dot(loc, wf[...], precision=_HIGH) + bf[...]
        g1 = jnp.maximum(
            jnp.dot(y, wg[...], precision=_HIGH) + bg[...], 0.0)
        u2_ref[...] = dv_ref[...] * g1

    full = lambda i: (0, 0)
    return pl.pallas_call(
        body,
        grid=(n // bn,),
        in_specs=[
            pl.BlockSpec((NC, bn, d), lambda i: (0, i, 0)),
            pl.BlockSpec((bn, d), lambda i: (i, 0)),
            pl.BlockSpec((bn, 1), lambda i: (i, 0)),
            pl.BlockSpec((d, d), full),
            pl.BlockSpec((1, d), full),
            pl.BlockSpec((d, d), full),
            pl.BlockSpec((1, d), full),
            pl.BlockSpec((d, d), full),
            pl.BlockSpec((1, d), full),
        ],
        out_specs=[
            pl.BlockSpec((bn, d), lambda i: (i, 0)),
            pl.BlockSpec((bn, d), lambda i: (i, 0)),
        ],
        out_shape=[
            jax.ShapeDtypeStruct((n, d), jnp.float32),
            jax.ShapeDtypeStruct((n, d), jnp.float32),
        ],
    )(s_p, u, dinv, w_local, b_local, w_g1, b_g1, w_f1, b_f)


def _tc_layer2(s_p, u2, dinv, w_g2, b_g2, w_f2, pre, bn=1000):
    n, d = u2.shape

    def body(s_ref, u2_ref, dv_ref, wg, bg, wf, pre_ref, out_ref):
        z = dv_ref[...] * (s_ref[0] + s_ref[1] + u2_ref[...])
        g2 = jnp.maximum(
            jnp.dot(z, wg[...], precision=_HIGH) + bg[...], 0.0)
        out_ref[...] = pre_ref[...] + jnp.dot(g2, wf[...], precision=_HIGH)

    full = lambda i: (0, 0)
    return pl.pallas_call(
        body,
        grid=(n // bn,),
        in_specs=[
            pl.BlockSpec((NC, bn, d), lambda i: (0, i, 0)),
            pl.BlockSpec((bn, d), lambda i: (i, 0)),
            pl.BlockSpec((bn, 1), lambda i: (i, 0)),
            pl.BlockSpec((d, d), full),
            pl.BlockSpec((1, d), full),
            pl.BlockSpec((d, d), full),
            pl.BlockSpec((bn, d), lambda i: (i, 0)),
        ],
        out_specs=pl.BlockSpec((bn, d), lambda i: (i, 0)),
        out_shape=jax.ShapeDtypeStruct((n, d), jnp.float32),
    )(s_p, u2, dinv, w_g2, b_g2, w_f2, pre)


def kernel(x, edge_index, W_local, b_local, W_g1, b_g1, W_g2, b_g2, W_fuse,
           b_fuse):
    n, d = x.shape
    h = W_local.shape[1]
    src = edge_index[0]
    dst = edge_index[1]
    w_f1 = W_fuse[:h]
    w_f2 = W_fuse[h:]
    b_l = b_local.reshape(1, -1)
    b_g1r = b_g1.reshape(1, -1)
    b_g2r = b_g2.reshape(1, -1)
    b_f = b_fuse.reshape(1, -1)

    deg_p = _sc_degree_hist(dst, n)
    dinv, u = _tc_prep(deg_p, x)
    s1 = _sc_aggregate(u, src, dst)
    pre, u2 = _tc_layer1(s1, u, dinv, W_local, b_l, W_g1, b_g1r, w_f1, b_f)
    s2 = _sc_aggregate(u2, src, dst)
    return _tc_layer2(s2, u2, dinv, W_g2, b_g2r, w_f2, pre)


# speedup vs baseline: 27.9165x; 2.0680x over previous
import functools

import jax
import jax.numpy as jnp
from jax import lax
from jax.experimental import pallas as pl
from jax.experimental.pallas import tpu as pltpu
from jax.experimental.pallas import tpu_sc as plsc

NC = 2
NS = 16
L = 16

CH = 80

_HIGH = jax.lax.Precision.HIGHEST


def _sc_mesh():
    return plsc.VectorSubcoreMesh(core_axis_name="c", subcore_axis_name="s")


def _sc_degree_hist(dst, n, d=128):
    (e,) = dst.shape
    epc = e // NC
    ept = epc // NS
    nch = ept // CH
    rpt = -(-n // NS)
    rpt = -(-rpt // 128) * 128
    npad = rpt * NS
    zr = 128
    assert epc * NC == e and ept * NS == epc and nch * CH == ept
    assert nch % 2 == 1

    @functools.partial(
        pl.kernel,
        mesh=_sc_mesh(),
        out_type=jax.ShapeDtypeStruct((NC, npad, d), jnp.float32),
        scratch_types=[
            pltpu.VMEM((CH,), jnp.int32),
            pltpu.VMEM((CH,), jnp.int32),
            pltpu.VMEM((CH, d), jnp.float32),
            pltpu.VMEM((zr, d), jnp.float32),
            pltpu.VMEM_SHARED((npad, d), jnp.float32),
            pltpu.SemaphoreType.DMA,
            pltpu.SemaphoreType.DMA,
        ],
    )
    def k(dst_hbm, out_hbm, ia_v, ib_v, ones_v, z_v, acc_sh, isa, isb):
        cid = lax.axis_index("c")
        sid = lax.axis_index("s")

        @pl.loop(0, CH)
        def _(r):
            @pl.loop(0, d, step=L)
            def _(c):
                ones_v[r, pl.ds(c, L)] = jnp.full((L,), 1.0, jnp.float32)

        @pl.loop(0, zr)
        def _(r):
            @pl.loop(0, d, step=L)
            def _(c):
                z_v[r, pl.ds(c, L)] = jnp.zeros((L,), jnp.float32)

        base = sid * rpt

        @pl.loop(0, rpt // zr)
        def _(j):
            pltpu.sync_copy(z_v, acc_sh.at[pl.ds(base + j * zr, zr)])

        ebase = cid * epc + sid * ept
        ca = pltpu.async_copy(dst_hbm.at[pl.ds(ebase, CH)], ia_v, isa)
        plsc.subcore_barrier()

        @pl.loop(0, (nch - 1) // 2)
        def _(jj):
            j0 = 2 * jj
            cb = pltpu.async_copy(
                dst_hbm.at[pl.ds(ebase + (j0 + 1) * CH, CH)], ib_v, isb)
            pltpu.make_async_copy(
                dst_hbm.at[pl.ds(ebase, CH)], ia_v, isa).wait()
            pltpu.sync_copy(ones_v, acc_sh.at[ia_v], add=True)
            ca2 = pltpu.async_copy(
                dst_hbm.at[pl.ds(ebase + (j0 + 2) * CH, CH)], ia_v, isa)
            pltpu.make_async_copy(
                dst_hbm.at[pl.ds(ebase, CH)], ib_v, isb).wait()
            pltpu.sync_copy(ones_v, acc_sh.at[ib_v], add=True)

        pltpu.make_async_copy(dst_hbm.at[pl.ds(ebase, CH)], ia_v, isa).wait()
        pltpu.sync_copy(ones_v, acc_sh.at[ia_v], add=True)

        plsc.subcore_barrier()
        pltpu.sync_copy(acc_sh.at[pl.ds(base, rpt)],
                        out_hbm.at[cid, pl.ds(base, rpt)])

    return k(dst)


def _sc_aggregate(u, src, dst):
    n, d = u.shape
    (e,) = src.shape
    epc = e // NC
    ept = epc // NS
    nch = ept // CH
    rpt = -(-n // NS)
    rpt = -(-rpt // 128) * 128
    npad = rpt * NS
    zr = 128
    assert epc * NC == e and ept * NS == epc and nch * CH == ept
    assert d % L == 0

    assert nch % 2 == 1

    @functools.partial(
        pl.kernel,
        mesh=_sc_mesh(),
        out_type=jax.ShapeDtypeStruct((NC, npad, d), jnp.float32),
        scratch_types=[
            pltpu.VMEM((ept,), jnp.int32),
            pltpu.VMEM((CH,), jnp.int32),
            pltpu.VMEM((CH,), jnp.int32),
            pltpu.VMEM((CH, d), jnp.float32),
            pltpu.VMEM((CH, d), jnp.float32),
            pltpu.VMEM_SHARED((npad, d), jnp.float32),
            pltpu.SemaphoreType.DMA,
            pltpu.SemaphoreType.DMA,
            pltpu.SemaphoreType.DMA,
            pltpu.SemaphoreType.DMA,
        ],
    )
    def k(u_hbm, src_hbm, dst_hbm, out_hbm, si_v, ia_v, ib_v, rowsa_v,
          rowsb_v, acc_sh, gsa, gsb, isa, isb):
        cid = lax.axis_index("c")
        sid = lax.axis_index("s")
        base = sid * rpt
        ebase = cid * epc + sid * ept

        @pl.loop(0, CH)
        def _(r):
            @pl.loop(0, d, step=L)
            def _(c):
                rowsa_v[r, pl.ds(c, L)] = jnp.zeros((L,), jnp.float32)

        @pl.loop(0, rpt // CH)
        def _(j):
            pltpu.sync_copy(rowsa_v, acc_sh.at[pl.ds(base + j * CH, CH)])

        pltpu.sync_copy(src_hbm.at[pl.ds(ebase, ept)], si_v)
        pltpu.async_copy(dst_hbm.at[pl.ds(ebase, CH)], ia_v, isa)
        pltpu.async_copy(u_hbm.at[si_v.at[pl.ds(0, CH)]], rowsa_v, gsa)
        plsc.subcore_barrier()

        @pl.loop(0, (nch - 1) // 2)
        def _(jj):
            j0 = 2 * jj
            pltpu.async_copy(
                dst_hbm.at[pl.ds(ebase + (j0 + 1) * CH, CH)], ib_v, isb)
            pltpu.async_copy(
                u_hbm.at[si_v.at[pl.ds((j0 + 1) * CH, CH)]], rowsb_v, gsb)
            pltpu.make_async_copy(
                u_hbm.at[si_v.at[pl.ds(0, CH)]], rowsa_v, gsa).wait()
            pltpu.make_async_copy(
                dst_hbm.at[pl.ds(ebase, CH)], ia_v, isa).wait()
            pltpu.sync_copy(rowsa_v, acc_sh.at[ia_v], add=True)
            pltpu.async_copy(
                dst_hbm.at[pl.ds(ebase + (j0 + 2) * CH, CH)], ia_v, isa)
            pltpu.async_copy(
                u_hbm.at[si_v.at[pl.ds((j0 + 2) * CH, CH)]], rowsa_v, gsa)
            pltpu.make_async_copy(
                u_hbm.at[si_v.at[pl.ds(0, CH)]], rowsb_v, gsb).wait()
            pltpu.make_async_copy(
                dst_hbm.at[pl.ds(ebase, CH)], ib_v, isb).wait()
            pltpu.sync_copy(rowsb_v, acc_sh.at[ib_v], add=True)

        pltpu.make_async_copy(
            u_hbm.at[si_v.at[pl.ds(0, CH)]], rowsa_v, gsa).wait()
        pltpu.make_async_copy(dst_hbm.at[pl.ds(ebase, CH)], ia_v, isa).wait()
        pltpu.sync_copy(rowsa_v, acc_sh.at[ia_v], add=True)

        plsc.subcore_barrier()
        pltpu.sync_copy(acc_sh.at[pl.ds(base, rpt)],
                        out_hbm.at[cid, pl.ds(base, rpt)])

    return k(u, src, dst)


def _tc_prep(deg_p, x, bn=1000):
    n, d = x.shape

    def body(dp_ref, x_ref, dinv_ref, u_ref):
        deg = dp_ref[0, :, 0:1] + dp_ref[1, :, 0:1] + 1.0
        dinv = lax.rsqrt(deg)
        dinv_ref[...] = dinv
        u_ref[...] = x_ref[...] * dinv

    return pl.pallas_call(
        body,
        grid=(n // bn,),
        in_specs=[
            pl.BlockSpec((NC, bn, deg_p.shape[2]), lambda i: (0, i, 0)),
            pl.BlockSpec((bn, d), lambda i: (i, 0)),
        ],
        out_specs=[
            pl.BlockSpec((bn, 1), lambda i: (i, 0)),
            pl.BlockSpec((bn, d), lambda i: (i, 0)),
        ],
        out_shape=[
            jax.ShapeDtypeStruct((n, 1), jnp.float32),
            jax.ShapeDtypeStruct((n, d), jnp.float32),
        ],
    )(deg_p, x)


def _tc_layer1(s_p, u, dinv, w_local, b_local, w_g1, b_g1, w_f1, b_f,
               bn=1000):
    n, d = u.shape

    def body(s_ref, u_ref, dv_ref, wl, bl, wg, bg, wf, bf, pre_ref, u2_ref):
        y = dv_ref[...] * (s_ref[0] + s_ref[1] + u_ref[...])
        loc = jnp.maximum(
            jnp.dot(y, wl[...], precision=_HIGH) + bl[...], 0.0)
        pre_ref[...] = jnp.dot(loc, wf[...], precision=_HIGH) + bf[...]
        g1 = jnp.maximum(
            jnp.dot(y, wg[...], precision=_HIGH) + bg[...], 0.0)
        u2_ref[...] = dv_ref[...] * g1

    full = lambda i: (0, 0)
    return pl.pallas_call(
        body,
        grid=(n // bn,),
        in_specs=[
            pl.BlockSpec((NC, bn, d), lambda i: (0, i, 0)),
            pl.BlockSpec((bn, d), lambda i: (i, 0)),
            pl.BlockSpec((bn, 1), lambda i: (i, 0)),
            pl.BlockSpec((d, d), full),
            pl.BlockSpec((1, d), full),
            pl.BlockSpec((d, d), full),
            pl.BlockSpec((1, d), full),
            pl.BlockSpec((d, d), full),
            pl.BlockSpec((1, d), full),
        ],
        out_specs=[
            pl.BlockSpec((bn, d), lambda i: (i, 0)),
            pl.BlockSpec((bn, d), lambda i: (i, 0)),
        ],
        out_shape=[
            jax.ShapeDtypeStruct((n, d), jnp.float32),
            jax.ShapeDtypeStruct((n, d), jnp.float32),
        ],
    )(s_p, u, dinv, w_local, b_local, w_g1, b_g1, w_f1, b_f)


def _tc_layer2(s_p, u2, dinv, w_g2, b_g2, w_f2, pre, bn=1000):
    n, d = u2.shape

    def body(s_ref, u2_ref, dv_ref, wg, bg, wf, pre_ref, out_ref):
        z = dv_ref[...] * (s_ref[0] + s_ref[1] + u2_ref[...])
        g2 = jnp.maximum(
            jnp.dot(z, wg[...], precision=_HIGH) + bg[...], 0.0)
        out_ref[...] = pre_ref[...] + jnp.dot(g2, wf[...], precision=_HIGH)

    full = lambda i: (0, 0)
    return pl.pallas_call(
        body,
        grid=(n // bn,),
        in_specs=[
            pl.BlockSpec((NC, bn, d), lambda i: (0, i, 0)),
            pl.BlockSpec((bn, d), lambda i: (i, 0)),
            pl.BlockSpec((bn, 1), lambda i: (i, 0)),
            pl.BlockSpec((d, d), full),
            pl.BlockSpec((1, d), full),
            pl.BlockSpec((d, d), full),
            pl.BlockSpec((bn, d), lambda i: (i, 0)),
        ],
        out_specs=pl.BlockSpec((bn, d), lambda i: (i, 0)),
        out_shape=jax.ShapeDtypeStruct((n, d), jnp.float32),
    )(s_p, u2, dinv, w_g2, b_g2, w_f2, pre)


def kernel(x, edge_index, W_local, b_local, W_g1, b_g1, W_g2, b_g2, W_fuse,
           b_fuse):
    n, d = x.shape
    h = W_local.shape[1]
    src = edge_index[0]
    dst = edge_index[1]
    w_f1 = W_fuse[:h]
    w_f2 = W_fuse[h:]
    b_l = b_local.reshape(1, -1)
    b_g1r = b_g1.reshape(1, -1)
    b_g2r = b_g2.reshape(1, -1)
    b_f = b_fuse.reshape(1, -1)

    deg_p = _sc_degree_hist(dst, n)
    dinv, u = _tc_prep(deg_p, x)
    s1 = _sc_aggregate(u, src, dst)
    pre, u2 = _tc_layer1(s1, u, dinv, W_local, b_l, W_g1, b_g1r, w_f1, b_f)
    s2 = _sc_aggregate(u2, src, dst)
    return _tc_layer2(s2, u2, dinv, W_g2, b_g2r, w_f2, pre)


# speedup vs baseline: 28.5442x; 1.0225x over previous
import functools

import jax
import jax.numpy as jnp
from jax import lax
from jax.experimental import pallas as pl
from jax.experimental.pallas import tpu as pltpu
from jax.experimental.pallas import tpu_sc as plsc

NC = 2
NS = 16
L = 16

CH = 80

_HIGH = jax.lax.Precision.HIGHEST


def _sc_mesh():
    return plsc.VectorSubcoreMesh(core_axis_name="c", subcore_axis_name="s")


def _sc_degree_hist(dst, n, d=128):
    (e,) = dst.shape
    epc = e // NC
    ept = epc // NS
    nch = ept // CH
    rpt = -(-n // NS)
    rpt = -(-rpt // 128) * 128
    npad = rpt * NS
    zr = 128
    assert epc * NC == e and ept * NS == epc and nch * CH == ept
    assert nch % 2 == 1

    @functools.partial(
        pl.kernel,
        mesh=_sc_mesh(),
        out_type=jax.ShapeDtypeStruct((NC, npad, d), jnp.float32),
        scratch_types=[
            pltpu.VMEM((CH,), jnp.int32),
            pltpu.VMEM((CH,), jnp.int32),
            pltpu.VMEM((CH, d), jnp.float32),
            pltpu.VMEM((zr, d), jnp.float32),
            pltpu.VMEM_SHARED((npad, d), jnp.float32),
            pltpu.SemaphoreType.DMA,
            pltpu.SemaphoreType.DMA,
        ],
    )
    def k(dst_hbm, out_hbm, ia_v, ib_v, ones_v, z_v, acc_sh, isa, isb):
        cid = lax.axis_index("c")
        sid = lax.axis_index("s")

        @pl.loop(0, CH)
        def _(r):
            @pl.loop(0, d, step=L)
            def _(c):
                ones_v[r, pl.ds(c, L)] = jnp.full((L,), 1.0, jnp.float32)

        @pl.loop(0, zr)
        def _(r):
            @pl.loop(0, d, step=L)
            def _(c):
                z_v[r, pl.ds(c, L)] = jnp.zeros((L,), jnp.float32)

        base = sid * rpt

        @pl.loop(0, rpt // zr)
        def _(j):
            pltpu.sync_copy(z_v, acc_sh.at[pl.ds(base + j * zr, zr)])

        ebase = cid * epc + sid * ept
        ca = pltpu.async_copy(dst_hbm.at[pl.ds(ebase, CH)], ia_v, isa)
        plsc.subcore_barrier()

        @pl.loop(0, (nch - 1) // 2)
        def _(jj):
            j0 = 2 * jj
            cb = pltpu.async_copy(
                dst_hbm.at[pl.ds(ebase + (j0 + 1) * CH, CH)], ib_v, isb)
            pltpu.make_async_copy(
                dst_hbm.at[pl.ds(ebase, CH)], ia_v, isa).wait()
            pltpu.sync_copy(ones_v, acc_sh.at[ia_v], add=True)
            ca2 = pltpu.async_copy(
                dst_hbm.at[pl.ds(ebase + (j0 + 2) * CH, CH)], ia_v, isa)
            pltpu.make_async_copy(
                dst_hbm.at[pl.ds(ebase, CH)], ib_v, isb).wait()
            pltpu.sync_copy(ones_v, acc_sh.at[ib_v], add=True)

        pltpu.make_async_copy(dst_hbm.at[pl.ds(ebase, CH)], ia_v, isa).wait()
        pltpu.sync_copy(ones_v, acc_sh.at[ia_v], add=True)

        plsc.subcore_barrier()
        pltpu.sync_copy(acc_sh.at[pl.ds(base, rpt)],
                        out_hbm.at[cid, pl.ds(base, rpt)])

    return k(dst)


def _sc_aggregate(u, src, dst):
    n, d = u.shape
    (e,) = src.shape
    epc = e // NC
    ept = epc // NS
    rpt = -(-n // NS)
    rpt = -(-rpt // 128) * 128
    npad = rpt * NS
    assert epc * NC == e and ept * NS == epc
    assert d % L == 0

    ch = CH // 2
    nch = ept // ch
    ns_ = 5
    nr = nch // ns_
    assert nch * ch == ept and nch % ns_ == 0 and nr >= 2

    @functools.partial(
        pl.kernel,
        mesh=_sc_mesh(),
        out_type=jax.ShapeDtypeStruct((NC, npad, d), jnp.float32),
        scratch_types=(
            [pltpu.VMEM((ept,), jnp.int32)]
            + [pltpu.VMEM((ch,), jnp.int32)] * ns_
            + [pltpu.VMEM((ch, d), jnp.float32)] * ns_
            + [pltpu.VMEM_SHARED((npad, d), jnp.float32)]
            + [pltpu.SemaphoreType.DMA] * (3 * ns_)
        ),
    )
    def k(u_hbm, src_hbm, dst_hbm, out_hbm, si_v, *rest):
        iv = rest[:ns_]
        rv = rest[ns_:2 * ns_]
        acc_sh = rest[2 * ns_]
        gs = rest[2 * ns_ + 1:2 * ns_ + 1 + ns_]
        isem = rest[2 * ns_ + 1 + ns_:2 * ns_ + 1 + 2 * ns_]
        ss = rest[2 * ns_ + 1 + 2 * ns_:2 * ns_ + 1 + 3 * ns_]
        cid = lax.axis_index("c")
        sid = lax.axis_index("s")
        base = sid * rpt
        ebase = cid * epc + sid * ept

        def idx_start(chunk, s):
            return pltpu.async_copy(
                dst_hbm.at[pl.ds(ebase + chunk * ch, ch)], iv[s], isem[s])

        def idx_wait(s):
            pltpu.make_async_copy(
                dst_hbm.at[pl.ds(ebase, ch)], iv[s], isem[s]).wait()

        def gat_start(chunk, s):
            return pltpu.async_copy(
                u_hbm.at[si_v.at[pl.ds(chunk * ch, ch)]], rv[s], gs[s])

        def gat_wait(s):
            pltpu.make_async_copy(
                u_hbm.at[si_v.at[pl.ds(0, ch)]], rv[s], gs[s]).wait()

        def sca_start(s):
            return pltpu.async_copy(rv[s], acc_sh.at[iv[s]], ss[s], add=True)

        def sca_wait(s):
            pltpu.make_async_copy(rv[s], acc_sh.at[iv[s]], ss[s]).wait()

        @pl.loop(0, ch)
        def _(r):
            @pl.loop(0, d, step=L)
            def _(c):
                rv[0][r, pl.ds(c, L)] = jnp.zeros((L,), jnp.float32)

        @pl.loop(0, rpt // ch)
        def _(j):
            pltpu.sync_copy(rv[0], acc_sh.at[pl.ds(base + j * ch, ch)])

        pltpu.sync_copy(src_hbm.at[pl.ds(ebase, ept)], si_v)
        for s in range(ns_):
            idx_start(s, s)
            gat_start(s, s)
        plsc.subcore_barrier()

        @pl.loop(0, nr - 1)
        def _(jj):
            for s in range(ns_):
                gat_wait(s)
                idx_wait(s)
                sca_start(s)
            nxt = ns_ * (jj + 1)
            for s in range(ns_):
                sca_wait(s)
                idx_start(nxt + s, s)
                gat_start(nxt + s, s)

        for s in range(ns_):
            gat_wait(s)
            idx_wait(s)
            sca_start(s)
        for s in range(ns_):
            sca_wait(s)

        plsc.subcore_barrier()
        pltpu.sync_copy(acc_sh.at[pl.ds(base, rpt)],
                        out_hbm.at[cid, pl.ds(base, rpt)])

    return k(u, src, dst)


def _tc_prep(deg_p, x, bn=1000):
    n, d = x.shape

    def body(dp_ref, x_ref, dinv_ref, u_ref):
        deg = dp_ref[0, :, 0:1] + dp_ref[1, :, 0:1] + 1.0
        dinv = lax.rsqrt(deg)
        dinv_ref[...] = dinv
        u_ref[...] = x_ref[...] * dinv

    return pl.pallas_call(
        body,
        grid=(n // bn,),
        in_specs=[
            pl.BlockSpec((NC, bn, deg_p.shape[2]), lambda i: (0, i, 0)),
            pl.BlockSpec((bn, d), lambda i: (i, 0)),
        ],
        out_specs=[
            pl.BlockSpec((bn, 1), lambda i: (i, 0)),
            pl.BlockSpec((bn, d), lambda i: (i, 0)),
        ],
        out_shape=[
            jax.ShapeDtypeStruct((n, 1), jnp.float32),
            jax.ShapeDtypeStruct((n, d), jnp.float32),
        ],
    )(deg_p, x)


def _tc_layer1(s_p, u, dinv, w_local, b_local, w_g1, b_g1, w_f1, b_f,
               bn=1000):
    n, d = u.shape

    def body(s_ref, u_ref, dv_ref, wl, bl, wg, bg, wf, bf, pre_ref, u2_ref):
        y = dv_ref[...] * (s_ref[0] + s_ref[1] + u_ref[...])
        loc = jnp.maximum(
            jnp.dot(y, wl[...], precision=_HIGH) + bl[...], 0.0)
        pre_ref[...] = jnp.dot(loc, wf[...], precision=_HIGH) + bf[...]
        g1 = jnp.maximum(
            jnp.dot(y, wg[...], precision=_HIGH) + bg[...], 0.0)
        u2_ref[...] = dv_ref[...] * g1

    full = lambda i: (0, 0)
    return pl.pallas_call(
        body,
        grid=(n // bn,),
        in_specs=[
            pl.BlockSpec((NC, bn, d), lambda i: (0, i, 0)),
            pl.BlockSpec((bn, d), lambda i: (i, 0)),
            pl.BlockSpec((bn, 1), lambda i: (i, 0)),
            pl.BlockSpec((d, d), full),
            pl.BlockSpec((1, d), full),
            pl.BlockSpec((d, d), full),
            pl.BlockSpec((1, d), full),
            pl.BlockSpec((d, d), full),
            pl.BlockSpec((1, d), full),
        ],
        out_specs=[
            pl.BlockSpec((bn, d), lambda i: (i, 0)),
            pl.BlockSpec((bn, d), lambda i: (i, 0)),
        ],
        out_shape=[
            jax.ShapeDtypeStruct((n, d), jnp.float32),
            jax.ShapeDtypeStruct((n, d), jnp.float32),
        ],
    )(s_p, u, dinv, w_local, b_local, w_g1, b_g1, w_f1, b_f)


def _tc_layer2(s_p, u2, dinv, w_g2, b_g2, w_f2, pre, bn=1000):
    n, d = u2.shape

    def body(s_ref, u2_ref, dv_ref, wg, bg, wf, pre_ref, out_ref):
        z = dv_ref[...] * (s_ref[0] + s_ref[1] + u2_ref[...])
        g2 = jnp.maximum(
            jnp.dot(z, wg[...], precision=_HIGH) + bg[...], 0.0)
        out_ref[...] = pre_ref[...] + jnp.dot(g2, wf[...], precision=_HIGH)

    full = lambda i: (0, 0)
    return pl.pallas_call(
        body,
        grid=(n // bn,),
        in_specs=[
            pl.BlockSpec((NC, bn, d), lambda i: (0, i, 0)),
            pl.BlockSpec((bn, d), lambda i: (i, 0)),
            pl.BlockSpec((bn, 1), lambda i: (i, 0)),
            pl.BlockSpec((d, d), full),
            pl.BlockSpec((1, d), full),
            pl.BlockSpec((d, d), full),
            pl.BlockSpec((bn, d), lambda i: (i, 0)),
        ],
        out_specs=pl.BlockSpec((bn, d), lambda i: (i, 0)),
        out_shape=jax.ShapeDtypeStruct((n, d), jnp.float32),
    )(s_p, u2, dinv, w_g2, b_g2, w_f2, pre)


def kernel(x, edge_index, W_local, b_local, W_g1, b_g1, W_g2, b_g2, W_fuse,
           b_fuse):
    n, d = x.shape
    h = W_local.shape[1]
    src = edge_index[0]
    dst = edge_index[1]
    w_f1 = W_fuse[:h]
    w_f2 = W_fuse[h:]
    b_l = b_local.reshape(1, -1)
    b_g1r = b_g1.reshape(1, -1)
    b_g2r = b_g2.reshape(1, -1)
    b_f = b_fuse.reshape(1, -1)

    deg_p = _sc_degree_hist(dst, n)
    dinv, u = _tc_prep(deg_p, x)
    s1 = _sc_aggregate(u, src, dst)
    pre, u2 = _tc_layer1(s1, u, dinv, W_local, b_l, W_g1, b_g1r, w_f1, b_f)
    s2 = _sc_aggregate(u2, src, dst)
    return _tc_layer2(s2, u2, dinv, W_g2, b_g2r, w_f2, pre)
